# baseline (speedup 1.0000x reference)
import jax
import jax.numpy as jnp
from jax import lax
from jax.experimental import pallas as pl
from jax.experimental.pallas import tpu as pltpu
from jax.experimental.pallas import tpu_sc as plsc

NN = 1000000
EMB = 64
B = 16384
NC, NS, L = 2, 16, 16
NW = NC * NS
SW = 512
NSLAB = 1953
TAIL0 = NSLAB * SW
LIST = 3264
GRP = 512
XCH = 2048
OUTR = 2 * B + 32


def _iota16():
    return lax.iota(jnp.int32, 16)


def _popcnt(m):
    return plsc.all_reduce_population_count(m)[0]


def _sc_body(x1_hbm, x2_hbm, w_hbm, embT_hbm, tail_hbm, out_hbm,
             w_v, xbuf_v, list_v, grp_v, slabA_v, slabB_v,
             stage0_v, stage1_v, semA, semB, ssem0, ssem1):
    wid = lax.axis_index("s") * NC + lax.axis_index("c")
    it16 = _iota16()
    slab0 = jnp.where(wid == 0, 0, 61 * wid + 1)
    nslab = jnp.where(wid == 0, 62, 61)
    base = slab0 * SW
    limit = nslab * SW + jnp.where(wid == NW - 1, EMB, 0)

    pltpu.sync_copy(w_hbm, w_v)
    z16 = jnp.zeros((L,), jnp.float32)
    for st in (stage0_v, stage1_v):
        for r in range(16):
            for c in range(8):
                st[r, pl.ds(c * 16, L)] = z16
    junk0 = 2 * B + it16
    junk1 = 2 * B + 16 + it16
    cp = pltpu.async_copy(stage0_v, out_hbm.at[junk0], ssem0)
    cp2 = pltpu.async_copy(stage1_v, out_hbm.at[junk1], ssem1)
    del cp, cp2

    def scan_side(x_hbm, side_bit, off0):
        off = off0
        for k in range(B // XCH):
            pltpu.sync_copy(x_hbm.at[pl.ds(k * XCH, XCH)], xbuf_v)

            def sv(v, off, k=k, side_bit=side_bit):
                e = xbuf_v[pl.ds(v * L, L)]
                rel = e - base
                m = (rel >= 0) & (rel < limit)
                pos = k * XCH + v * L + it16
                ent = rel | side_bit | (pos << 16)
                inc = m.astype(jnp.int32)
                c = plsc.cumsum(inc)
                dest = jnp.minimum(off + c - inc, LIST - 1)
                plsc.store_scatter(list_v, [dest], ent, mask=m)
                return off + c[L - 1]

            off = lax.fori_loop(0, XCH // L, sv, off, unroll=4)
        return off

    nlist = scan_side(x2_hbm, 1 << 15, scan_side(x1_hbm, 0, 0))
    nlv = (nlist + L - 1) // L

    def compute_slab(si, slab_ref, col_off):
        def cs(u, off, slab_si=si):
            for j in range(4):
                v = u * 4 + j
                e = list_v[pl.ds(v * L, L)]
                rel = e & 0x7FFF
                m = ((rel >> 9) == slab_si) & ((v * L + it16) < nlist)
                inc = m.astype(jnp.int32)
                c = plsc.cumsum(inc)
                dest = jnp.minimum(off + c - inc, GRP - 1)
                plsc.store_scatter(grp_v, [dest], e, mask=m)
                off = off + c[L - 1]
            return off

        cnt = lax.fori_loop(0, (nlist + 63) // 64, cs, 0)
        ngrp = (cnt + L - 1) // L

        def build(g, stage_ref, ssem, jrow):
            eg = grp_v[pl.ds(g * L, L)]
            valid = (g * L + it16) < cnt
            col = jnp.where(valid, (eg & 511) + col_off, 0)
            side = (eg >> 15) & 1
            pos = jnp.where(valid, eg >> 16, 0)
            wv = plsc.load_gather(w_v, [pos])
            wsel = jnp.where((side == 0) & valid, wv, jnp.ones((L,), jnp.float32))
            pltpu.make_async_copy(stage_ref, out_hbm.at[jrow], ssem).wait()
            for d in range(EMB):
                dvec = jnp.full((L,), d, jnp.int32)
                fv = plsc.load_gather(slab_ref, [dvec, col])
                plsc.store_scatter(stage_ref, [it16, dvec], fv * wsel)
            drow = jnp.where(valid, pos + side * B, jrow)
            pltpu.async_copy(stage_ref, out_hbm.at[drow], ssem)

        def gpair(t, carry):
            build(2 * t, stage0_v, ssem0, junk0)

            @pl.when(2 * t + 1 < ngrp)
            def _():
                build(2 * t + 1, stage1_v, ssem1, junk1)

            return carry

        lax.fori_loop(0, (ngrp + 1) // 2, gpair, 0)

    def start(s_local, slab_ref, sem):
        src = embT_hbm.at[:, pl.ds((slab0 + s_local) * SW, SW)]
        return pltpu.async_copy(src, slab_ref, sem)

    def wait(s_local, slab_ref, sem):
        src = embT_hbm.at[:, pl.ds((slab0 + s_local) * SW, SW)]
        pltpu.make_async_copy(src, slab_ref, sem).wait()

    start(0, slabA_v, semA)

    @pl.when(1 < nslab)
    def _():
        start(1, slabB_v, semB)

    def pair(t, carry):
        s0 = 2 * t
        wait(s0, slabA_v, semA)
        compute_slab(s0, slabA_v, 0)

        @pl.when(s0 + 2 < nslab)
        def _():
            start(s0 + 2, slabA_v, semA)

        @pl.when(s0 + 1 < nslab)
        def _():
            wait(s0 + 1, slabB_v, semB)
            compute_slab(s0 + 1, slabB_v, 0)

            @pl.when(s0 + 3 < nslab)
            def _():
                start(s0 + 3, slabB_v, semB)

        return carry

    lax.fori_loop(0, 31, pair, 0)

    @pl.when(wid == NW - 1)
    def _():
        pltpu.sync_copy(tail_hbm, slabA_v.at[:, pl.ds(0, 2 * EMB)])
        compute_slab(61, slabA_v, EMB)

    pltpu.make_async_copy(stage0_v, out_hbm.at[junk0], ssem0).wait()
    pltpu.make_async_copy(stage1_v, out_hbm.at[junk1], ssem1).wait()


def _gather_rows(x1, x2, w, emb):
    embT = emb.T
    tail = lax.slice(embT, (0, NN - 2 * EMB), (EMB, NN))
    mesh = plsc.VectorSubcoreMesh(core_axis_name="c", subcore_axis_name="s")
    return pl.kernel(
        _sc_body,
        out_type=jax.ShapeDtypeStruct((OUTR, 2 * EMB), jnp.float32),
        mesh=mesh,
        compiler_params=pltpu.CompilerParams(needs_layout_passes=False),
        scratch_types=[
            pltpu.VMEM((B,), jnp.float32),
            pltpu.VMEM((XCH,), jnp.int32),
            pltpu.VMEM((LIST,), jnp.int32),
            pltpu.VMEM((GRP,), jnp.int32),
            pltpu.VMEM((EMB, SW), jnp.float32),
            pltpu.VMEM((EMB, SW), jnp.float32),
            pltpu.VMEM((L, 2 * EMB), jnp.float32),
            pltpu.VMEM((L, 2 * EMB), jnp.float32),
            pltpu.SemaphoreType.DMA,
            pltpu.SemaphoreType.DMA,
            pltpu.SemaphoreType.DMA,
            pltpu.SemaphoreType.DMA,
        ],
    )(x1, x2, w, embT, tail)


def _tc_body(p_ref, o_ref):
    e1 = p_ref[pl.ds(0, B), :]
    e2 = p_ref[pl.ds(B, B), :]
    x = jnp.sum(e1 * e2, axis=1, keepdims=True)
    ls = jnp.minimum(x, 0.0) - jnp.log1p(jnp.exp(-jnp.abs(x)))
    o_ref[0, 0] = -jnp.sum(ls) / B


def _loss(rows):
    return pl.pallas_call(
        _tc_body,
        out_shape=jax.ShapeDtypeStruct((1, 1), jnp.float32),
        in_specs=[pl.BlockSpec(memory_space=pltpu.VMEM)],
        out_specs=pl.BlockSpec(memory_space=pltpu.SMEM),
    )(rows)


@jax.jit
def kernel(x1, x2, w, emb):
    return _loss(_gather_rows(x1, x2, w, emb))[0, 0]


# speedup vs baseline: 1.1531x; 1.1531x over previous
import jax
import jax.numpy as jnp
from jax import lax
from jax.experimental import pallas as pl
from jax.experimental.pallas import tpu as pltpu
from jax.experimental.pallas import tpu_sc as plsc

NN = 1000000
EMB = 64
B = 16384
NC, NS, L = 2, 16, 16
NW = NC * NS
SW = 512
NSLAB = 1953
TAIL0 = NSLAB * SW
LIST = 3264
GRP = 512
XCH = 2048
OUTR = 2 * B + 128


def _iota16():
    return lax.iota(jnp.int32, 16)


def _popcnt(m):
    return plsc.all_reduce_population_count(m)[0]


def _sc_body(x1_hbm, x2_hbm, w_hbm, embT_hbm, tail_hbm, out_hbm,
             w_v, xbuf_v, list_v, grp_v, slabA_v, slabB_v,
             st0, st1, st2, st3, st4, st5, st6, st7,
             semA, semB, ss0, ss1, ss2, ss3, ss4, ss5, ss6, ss7):
    wid = lax.axis_index("s") * NC + lax.axis_index("c")
    it16 = _iota16()
    slab0 = jnp.where(wid == 0, 0, 61 * wid + 1)
    nslab = jnp.where(wid == 0, 62, 61)
    base = slab0 * SW
    limit = nslab * SW + jnp.where(wid == NW - 1, EMB, 0)

    pltpu.sync_copy(w_hbm, w_v)
    z16 = jnp.zeros((L,), jnp.float32)
    stages = (st0, st1, st2, st3, st4, st5, st6, st7)
    ssems = (ss0, ss1, ss2, ss3, ss4, ss5, ss6, ss7)
    junks = tuple(2 * B + 16 * j + it16 for j in range(8))
    for j, st in enumerate(stages):
        for r in range(16):
            for c in range(8):
                st[r, pl.ds(c * 16, L)] = z16
        pltpu.async_copy(st, out_hbm.at[junks[j]], ssems[j])

    def scan_side(x_hbm, side_bit, off0):
        off = off0
        for k in range(B // XCH):
            pltpu.sync_copy(x_hbm.at[pl.ds(k * XCH, XCH)], xbuf_v)

            def sv(v, off, k=k, side_bit=side_bit):
                e = xbuf_v[pl.ds(v * L, L)]
                rel = e - base
                m = (rel >= 0) & (rel < limit)
                pos = k * XCH + v * L + it16
                ent = rel | side_bit | (pos << 16)
                inc = m.astype(jnp.int32)
                c = plsc.cumsum(inc)
                dest = jnp.minimum(off + c - inc, LIST - 1)
                plsc.store_scatter(list_v, [dest], ent, mask=m)
                return off + c[L - 1]

            off = lax.fori_loop(0, XCH // L, sv, off, unroll=4)
        return off

    nlist = scan_side(x2_hbm, 1 << 15, scan_side(x1_hbm, 0, 0))
    nlv = (nlist + L - 1) // L

    def compute_slab(si, slab_ref, col_off, bufset):
        def cs(u, off, slab_si=si):
            for j in range(4):
                v = u * 4 + j
                e = list_v[pl.ds(v * L, L)]
                rel = e & 0x7FFF
                m = ((rel >> 9) == slab_si) & ((v * L + it16) < nlist)
                inc = m.astype(jnp.int32)
                c = plsc.cumsum(inc)
                dest = jnp.minimum(off + c - inc, GRP - 1)
                plsc.store_scatter(grp_v, [dest], e, mask=m)
                off = off + c[L - 1]
            return off

        cnt = lax.fori_loop(0, (nlist + 63) // 64, cs, 0)
        ngrp = (cnt + L - 1) // L

        def build(g, stage_ref, ssem, jrow):
            eg = grp_v[pl.ds(g * L, L)]
            valid = (g * L + it16) < cnt
            col = jnp.where(valid, (eg & 511) + col_off, 0)
            side = (eg >> 15) & 1
            pos = jnp.where(valid, eg >> 16, 0)
            wv = plsc.load_gather(w_v, [pos])
            wsel = jnp.where((side == 0) & valid, wv, jnp.ones((L,), jnp.float32))
            pltpu.make_async_copy(stage_ref, out_hbm.at[jrow], ssem).wait()

            def dblk(i, carry):
                for j in range(8):
                    dvec = jnp.full((L,), 0, jnp.int32) + (i * 8 + j)
                    fv = plsc.load_gather(slab_ref, [dvec, col])
                    plsc.store_scatter(stage_ref, [it16, dvec], fv * wsel)
                return carry

            lax.fori_loop(0, EMB // 8, dblk, 0)
            drow = jnp.where(valid, pos + side * B, jrow)
            pltpu.async_copy(stage_ref, out_hbm.at[drow], ssem)

        def gquad(t, carry):
            build(4 * t, stages[bufset], ssems[bufset], junks[bufset])
            for j in range(1, 4):
                @pl.when(4 * t + j < ngrp)
                def _(j=j):
                    build(4 * t + j, stages[bufset + j],
                          ssems[bufset + j], junks[bufset + j])
            return carry

        lax.fori_loop(0, (ngrp + 3) // 4, gquad, 0)

    def start(s_local, slab_ref, sem):
        src = embT_hbm.at[:, pl.ds((slab0 + s_local) * SW, SW)]
        return pltpu.async_copy(src, slab_ref, sem)

    def wait(s_local, slab_ref, sem):
        src = embT_hbm.at[:, pl.ds((slab0 + s_local) * SW, SW)]
        pltpu.make_async_copy(src, slab_ref, sem).wait()

    start(0, slabA_v, semA)

    @pl.when(1 < nslab)
    def _():
        start(1, slabB_v, semB)

    def pair(t, carry):
        s0 = 2 * t
        wait(s0, slabA_v, semA)
        compute_slab(s0, slabA_v, 0, 0)

        @pl.when(s0 + 2 < nslab)
        def _():
            start(s0 + 2, slabA_v, semA)

        @pl.when(s0 + 1 < nslab)
        def _():
            wait(s0 + 1, slabB_v, semB)
            compute_slab(s0 + 1, slabB_v, 0, 4)

            @pl.when(s0 + 3 < nslab)
            def _():
                start(s0 + 3, slabB_v, semB)

        return carry

    lax.fori_loop(0, 31, pair, 0)

    @pl.when(wid == NW - 1)
    def _():
        pltpu.sync_copy(tail_hbm, slabA_v.at[:, pl.ds(0, 2 * EMB)])
        compute_slab(61, slabA_v, EMB, 0)

    for j in range(8):
        pltpu.make_async_copy(stages[j], out_hbm.at[junks[j]], ssems[j]).wait()


def _gather_rows(x1, x2, w, emb):
    embT = emb.T
    tail = lax.slice(embT, (0, NN - 2 * EMB), (EMB, NN))
    mesh = plsc.VectorSubcoreMesh(core_axis_name="c", subcore_axis_name="s")
    return pl.kernel(
        _sc_body,
        out_type=jax.ShapeDtypeStruct((OUTR, 2 * EMB), jnp.float32),
        mesh=mesh,
        compiler_params=pltpu.CompilerParams(needs_layout_passes=False),
        scratch_types=[
            pltpu.VMEM((B,), jnp.float32),
            pltpu.VMEM((XCH,), jnp.int32),
            pltpu.VMEM((LIST,), jnp.int32),
            pltpu.VMEM((GRP,), jnp.int32),
            pltpu.VMEM((EMB, SW), jnp.float32),
            pltpu.VMEM((EMB, SW), jnp.float32),
        ] + [pltpu.VMEM((L, 2 * EMB), jnp.float32)] * 8
          + [pltpu.SemaphoreType.DMA] * 10,
    )(x1, x2, w, embT, tail)


def _tc_body(p_ref, o_ref):
    e1 = p_ref[pl.ds(0, B), :]
    e2 = p_ref[pl.ds(B, B), :]
    x = jnp.sum(e1 * e2, axis=1, keepdims=True)
    ls = jnp.minimum(x, 0.0) - jnp.log1p(jnp.exp(-jnp.abs(x)))
    o_ref[0, 0] = -jnp.sum(ls) / B


def _loss(rows):
    return pl.pallas_call(
        _tc_body,
        out_shape=jax.ShapeDtypeStruct((1, 1), jnp.float32),
        in_specs=[pl.BlockSpec(memory_space=pltpu.VMEM)],
        out_specs=pl.BlockSpec(memory_space=pltpu.SMEM),
    )(rows)


@jax.jit
def kernel(x1, x2, w, emb):
    return _loss(_gather_rows(x1, x2, w, emb))[0, 0]


# speedup vs baseline: 1.1703x; 1.0149x over previous
import jax
import jax.numpy as jnp
from jax import lax
from jax.experimental import pallas as pl
from jax.experimental.pallas import tpu as pltpu
from jax.experimental.pallas import tpu_sc as plsc

NN = 1000000
EMB = 64
B = 16384
NC, NS, L = 2, 16, 16
NW = NC * NS
SW = 512
NSLAB = 1953
TAIL0 = NSLAB * SW
LIST = 3264
GRP = 512
XCH = 2048
OUTR = 2 * B + 128


def _iota16():
    return lax.iota(jnp.int32, 16)


def _popcnt(m):
    return plsc.all_reduce_population_count(m)[0]


def _sc_body(x1_hbm, x2_hbm, w_hbm, embT_hbm, tail_hbm, out_hbm,
             w_v, xbuf_v, list_v, grp_v, slabA_v, slabB_v,
             st0, st1, st2, st3, st4, st5, st6, st7,
             semA, semB, ss0, ss1, ss2, ss3, ss4, ss5, ss6, ss7):
    wid = lax.axis_index("s") * NC + lax.axis_index("c")
    it16 = _iota16()
    slab0 = jnp.where(wid == 0, 0, 61 * wid + 1)
    nslab = jnp.where(wid == 0, 62, 61)
    base = slab0 * SW
    limit = nslab * SW + jnp.where(wid == NW - 1, EMB, 0)

    pltpu.sync_copy(w_hbm, w_v)
    z16 = jnp.zeros((L,), jnp.float32)
    stages = (st0, st1, st2, st3, st4, st5, st6, st7)
    ssems = (ss0, ss1, ss2, ss3, ss4, ss5, ss6, ss7)
    junks = tuple(2 * B + 16 * j + it16 for j in range(8))
    for j, st in enumerate(stages):
        for r in range(16):
            for c in range(8):
                st[r, pl.ds(c * 16, L)] = z16
        pltpu.async_copy(st, out_hbm.at[junks[j]], ssems[j])

    def start(s_local, slab_ref, sem):
        src = embT_hbm.at[:, pl.ds((slab0 + s_local) * SW, SW)]
        return pltpu.async_copy(src, slab_ref, sem)

    def wait(s_local, slab_ref, sem):
        src = embT_hbm.at[:, pl.ds((slab0 + s_local) * SW, SW)]
        pltpu.make_async_copy(src, slab_ref, sem).wait()

    start(0, slabA_v, semA)
    start(1, slabB_v, semB)

    def scan_side(x_hbm, side_bit, off0):
        off = off0
        for k in range(B // XCH):
            pltpu.sync_copy(x_hbm.at[pl.ds(k * XCH, XCH)], xbuf_v)

            def sv(v, off, k=k, side_bit=side_bit):
                e = xbuf_v[pl.ds(v * L, L)]
                rel = e - base
                m = (rel >= 0) & (rel < limit)
                pos = k * XCH + v * L + it16
                ent = rel | side_bit | (pos << 16)
                inc = m.astype(jnp.int32)
                c = plsc.cumsum(inc)
                dest = jnp.minimum(off + c - inc, LIST - 1)
                plsc.store_scatter(list_v, [dest], ent, mask=m)
                return off + c[L - 1]

            off = lax.fori_loop(0, XCH // L, sv, off, unroll=4)
        return off

    nlist = scan_side(x2_hbm, 1 << 15, scan_side(x1_hbm, 0, 0))
    nlv = (nlist + L - 1) // L

    def compute_slab(si, slab_ref, col_off, bufset):
        def cs(u, off, slab_si=si):
            for j in range(4):
                v = u * 4 + j
                e = list_v[pl.ds(v * L, L)]
                rel = e & 0x7FFF
                m = ((rel >> 9) == slab_si) & ((v * L + it16) < nlist)
                inc = m.astype(jnp.int32)
                c = plsc.cumsum(inc)
                dest = jnp.minimum(off + c - inc, GRP - 1)
                plsc.store_scatter(grp_v, [dest], e, mask=m)
                off = off + c[L - 1]
            return off

        cnt = lax.fori_loop(0, (nlist + 63) // 64, cs, 0)
        ngrp = (cnt + L - 1) // L

        def build(g, stage_ref, ssem, jrow):
            eg = grp_v[pl.ds(g * L, L)]
            valid = (g * L + it16) < cnt
            col = jnp.where(valid, (eg & 511) + col_off, 0)
            side = (eg >> 15) & 1
            pos = jnp.where(valid, eg >> 16, 0)
            wv = plsc.load_gather(w_v, [pos])
            wsel = jnp.where((side == 0) & valid, wv, jnp.ones((L,), jnp.float32))
            pltpu.make_async_copy(stage_ref, out_hbm.at[jrow], ssem).wait()

            def dblk(i, carry):
                for j in range(8):
                    dvec = jnp.full((L,), 0, jnp.int32) + (i * 8 + j)
                    fv = plsc.load_gather(slab_ref, [dvec, col])
                    plsc.store_scatter(stage_ref, [it16, dvec], fv * wsel)
                return carry

            lax.fori_loop(0, EMB // 8, dblk, 0)
            drow = jnp.where(valid, pos + side * B, jrow)
            pltpu.async_copy(stage_ref, out_hbm.at[drow], ssem)

        def gquad(t, carry):
            build(4 * t, stages[bufset], ssems[bufset], junks[bufset])
            for j in range(1, 4):
                @pl.when(4 * t + j < ngrp)
                def _(j=j):
                    build(4 * t + j, stages[bufset + j],
                          ssems[bufset + j], junks[bufset + j])
            return carry

        lax.fori_loop(0, (ngrp + 3) // 4, gquad, 0)

    def pair(t, carry):
        s0 = 2 * t
        wait(s0, slabA_v, semA)
        compute_slab(s0, slabA_v, 0, 0)

        @pl.when(s0 + 2 < nslab)
        def _():
            start(s0 + 2, slabA_v, semA)

        @pl.when(s0 + 1 < nslab)
        def _():
            wait(s0 + 1, slabB_v, semB)
            compute_slab(s0 + 1, slabB_v, 0, 4)

            @pl.when(s0 + 3 < nslab)
            def _():
                start(s0 + 3, slabB_v, semB)

        return carry

    lax.fori_loop(0, 31, pair, 0)

    @pl.when(wid == NW - 1)
    def _():
        pltpu.sync_copy(tail_hbm, slabA_v.at[:, pl.ds(0, 2 * EMB)])
        compute_slab(61, slabA_v, EMB, 0)

    for j in range(8):
        pltpu.make_async_copy(stages[j], out_hbm.at[junks[j]], ssems[j]).wait()


def _gather_rows(x1, x2, w, emb):
    embT = emb.T
    tail = lax.slice(embT, (0, NN - 2 * EMB), (EMB, NN))
    mesh = plsc.VectorSubcoreMesh(core_axis_name="c", subcore_axis_name="s")
    return pl.kernel(
        _sc_body,
        out_type=jax.ShapeDtypeStruct((OUTR, 2 * EMB), jnp.float32),
        mesh=mesh,
        compiler_params=pltpu.CompilerParams(needs_layout_passes=False),
        scratch_types=[
            pltpu.VMEM((B,), jnp.float32),
            pltpu.VMEM((XCH,), jnp.int32),
            pltpu.VMEM((LIST,), jnp.int32),
            pltpu.VMEM((GRP,), jnp.int32),
            pltpu.VMEM((EMB, SW), jnp.float32),
            pltpu.VMEM((EMB, SW), jnp.float32),
        ] + [pltpu.VMEM((L, 2 * EMB), jnp.float32)] * 8
          + [pltpu.SemaphoreType.DMA] * 10,
    )(x1, x2, w, embT, tail)


def _tc_body(p_ref, o_ref):
    e1 = p_ref[pl.ds(0, B), :]
    e2 = p_ref[pl.ds(B, B), :]
    x = jnp.sum(e1 * e2, axis=1, keepdims=True)
    ls = jnp.minimum(x, 0.0) - jnp.log1p(jnp.exp(-jnp.abs(x)))
    o_ref[0, 0] = -jnp.sum(ls) / B


def _loss(rows):
    return pl.pallas_call(
        _tc_body,
        out_shape=jax.ShapeDtypeStruct((1, 1), jnp.float32),
        in_specs=[pl.BlockSpec(memory_space=pltpu.VMEM)],
        out_specs=pl.BlockSpec(memory_space=pltpu.SMEM),
    )(rows)


@jax.jit
def kernel(x1, x2, w, emb):
    return _loss(_gather_rows(x1, x2, w, emb))[0, 0]


# speedup vs baseline: 1.1848x; 1.0124x over previous
import jax
import jax.numpy as jnp
from jax import lax
from jax.experimental import pallas as pl
from jax.experimental.pallas import tpu as pltpu
from jax.experimental.pallas import tpu_sc as plsc

NN = 1000000
EMB = 64
B = 16384
NC, NS, L = 2, 16, 16
NW = NC * NS
SW = 512
NSLAB = 1953
TAIL0 = NSLAB * SW
LIST = 3264
GRP = 512
XCH = 2048
OUTR = 2 * B + 128


def _iota16():
    return lax.iota(jnp.int32, 16)


def _popcnt(m):
    return plsc.all_reduce_population_count(m)[0]


def _sc_body(x1_hbm, x2_hbm, w_hbm, embT_hbm, tail_hbm, out_hbm,
             w_v, xbuf_v, list_v, grp_v, slabA_v, slabB_v,
             st0, st1, st2, st3, st4, st5, st6, st7,
             semA, semB, ss0, ss1, ss2, ss3, ss4, ss5, ss6, ss7):
    wid = lax.axis_index("s") * NC + lax.axis_index("c")
    it16 = _iota16()
    slab0 = jnp.where(wid == 0, 0, 61 * wid + 1)
    nslab = jnp.where(wid == 0, 62, 61)
    base = slab0 * SW
    limit = nslab * SW + jnp.where(wid == NW - 1, EMB, 0)

    pltpu.sync_copy(w_hbm, w_v)
    z16 = jnp.zeros((L,), jnp.float32)
    stages = (st0, st1, st2, st3, st4, st5, st6, st7)
    ssems = (ss0, ss1, ss2, ss3, ss4, ss5, ss6, ss7)
    junks = tuple(2 * B + 16 * j + it16 for j in range(8))
    for j, st in enumerate(stages):
        for r in range(16):
            for c in range(8):
                st[r, pl.ds(c * 16, L)] = z16
        pltpu.async_copy(st, out_hbm.at[junks[j]], ssems[j])

    def start(s_local, slab_ref, sem):
        src = embT_hbm.at[:, pl.ds((slab0 + s_local) * SW, SW)]
        return pltpu.async_copy(src, slab_ref, sem)

    def wait(s_local, slab_ref, sem):
        src = embT_hbm.at[:, pl.ds((slab0 + s_local) * SW, SW)]
        pltpu.make_async_copy(src, slab_ref, sem).wait()

    start(0, slabA_v, semA)
    start(1, slabB_v, semB)

    def scan_side(x_hbm, side_bit, off0):
        off = off0
        for k in range(B // XCH):
            pltpu.sync_copy(x_hbm.at[pl.ds(k * XCH, XCH)], xbuf_v)

            def sv(v, off, k=k, side_bit=side_bit):
                e = xbuf_v[pl.ds(v * L, L)]
                rel = e - base
                m = (rel >= 0) & (rel < limit)
                pos = k * XCH + v * L + it16
                ent = rel | side_bit | (pos << 16)
                inc = m.astype(jnp.int32)
                c = plsc.cumsum(inc)
                dest = jnp.minimum(off + c - inc, LIST - 1)
                plsc.store_scatter(list_v, [dest], ent, mask=m)
                return off + c[L - 1]

            off = lax.fori_loop(0, XCH // L, sv, off, unroll=4)
        return off

    nlist = scan_side(x2_hbm, 1 << 15, scan_side(x1_hbm, 0, 0))
    sent = jnp.full((L,), 0x7FFF, jnp.int32)

    def pad(v, carry):
        plsc.store_scatter(list_v, [jnp.minimum(nlist + v * L + it16, LIST - 1)],
                           sent, mask=sent > 0)
        return carry

    lax.fori_loop(0, 4, pad, 0)

    def compute_slab(si, slab_ref, col_off, bufset):
        def cs(u, off, slab_si=si):
            for j in range(4):
                v = u * 4 + j
                e = list_v[pl.ds(v * L, L)]
                rel = e & 0x7FFF
                m = (rel >> 9) == slab_si
                inc = m.astype(jnp.int32)
                c = plsc.cumsum(inc)
                dest = jnp.minimum(off + c - inc, GRP - 1)
                plsc.store_scatter(grp_v, [dest], e, mask=m)
                off = off + c[L - 1]
            return off

        cnt = lax.fori_loop(0, (nlist + 63) // 64, cs, 0)
        ngrp = (cnt + L - 1) // L

        def build(g, stage_ref, ssem, jrow):
            eg = grp_v[pl.ds(g * L, L)]
            valid = (g * L + it16) < cnt
            col = jnp.where(valid, (eg & 511) + col_off, 0)
            side = (eg >> 15) & 1
            pos = jnp.where(valid, eg >> 16, 0)
            wv = plsc.load_gather(w_v, [pos])
            wsel = jnp.where((side == 0) & valid, wv, jnp.ones((L,), jnp.float32))
            pltpu.make_async_copy(stage_ref, out_hbm.at[jrow], ssem).wait()

            def dblk(i, carry):
                for j in range(8):
                    dvec = jnp.full((L,), 0, jnp.int32) + (i * 8 + j)
                    fv = plsc.load_gather(slab_ref, [dvec, col])
                    plsc.store_scatter(stage_ref, [it16, dvec], fv * wsel)
                return carry

            lax.fori_loop(0, EMB // 8, dblk, 0)
            drow = jnp.where(valid, pos + side * B, jrow)
            pltpu.async_copy(stage_ref, out_hbm.at[drow], ssem)

        def gquad(t, carry):
            build(4 * t, stages[bufset], ssems[bufset], junks[bufset])
            for j in range(1, 4):
                @pl.when(4 * t + j < ngrp)
                def _(j=j):
                    build(4 * t + j, stages[bufset + j],
                          ssems[bufset + j], junks[bufset + j])
            return carry

        lax.fori_loop(0, (ngrp + 3) // 4, gquad, 0)

    def pair(t, carry):
        s0 = 2 * t
        wait(s0, slabA_v, semA)
        compute_slab(s0, slabA_v, 0, 0)

        @pl.when(s0 + 2 < nslab)
        def _():
            start(s0 + 2, slabA_v, semA)

        @pl.when(s0 + 1 < nslab)
        def _():
            wait(s0 + 1, slabB_v, semB)
            compute_slab(s0 + 1, slabB_v, 0, 4)

            @pl.when(s0 + 3 < nslab)
            def _():
                start(s0 + 3, slabB_v, semB)

        return carry

    lax.fori_loop(0, 31, pair, 0)

    @pl.when(wid == NW - 1)
    def _():
        pltpu.sync_copy(tail_hbm, slabA_v.at[:, pl.ds(0, 2 * EMB)])
        compute_slab(61, slabA_v, EMB, 0)

    for j in range(8):
        pltpu.make_async_copy(stages[j], out_hbm.at[junks[j]], ssems[j]).wait()


def _gather_rows(x1, x2, w, emb):
    embT = emb.T
    tail = lax.slice(embT, (0, NN - 2 * EMB), (EMB, NN))
    mesh = plsc.VectorSubcoreMesh(core_axis_name="c", subcore_axis_name="s")
    return pl.kernel(
        _sc_body,
        out_type=jax.ShapeDtypeStruct((OUTR, 2 * EMB), jnp.float32),
        mesh=mesh,
        compiler_params=pltpu.CompilerParams(needs_layout_passes=False),
        scratch_types=[
            pltpu.VMEM((B,), jnp.float32),
            pltpu.VMEM((XCH,), jnp.int32),
            pltpu.VMEM((LIST,), jnp.int32),
            pltpu.VMEM((GRP,), jnp.int32),
            pltpu.VMEM((EMB, SW), jnp.float32),
            pltpu.VMEM((EMB, SW), jnp.float32),
        ] + [pltpu.VMEM((L, 2 * EMB), jnp.float32)] * 8
          + [pltpu.SemaphoreType.DMA] * 10,
    )(x1, x2, w, embT, tail)


def _tc_body(p_ref, o_ref):
    e1 = p_ref[pl.ds(0, B), :]
    e2 = p_ref[pl.ds(B, B), :]
    x = jnp.sum(e1 * e2, axis=1, keepdims=True)
    ls = jnp.minimum(x, 0.0) - jnp.log1p(jnp.exp(-jnp.abs(x)))
    o_ref[0, 0] = -jnp.sum(ls) / B


def _loss(rows):
    return pl.pallas_call(
        _tc_body,
        out_shape=jax.ShapeDtypeStruct((1, 1), jnp.float32),
        in_specs=[pl.BlockSpec(memory_space=pltpu.VMEM)],
        out_specs=pl.BlockSpec(memory_space=pltpu.SMEM),
    )(rows)


@jax.jit
def kernel(x1, x2, w, emb):
    return _loss(_gather_rows(x1, x2, w, emb))[0, 0]


# speedup vs baseline: 1.2176x; 1.0277x over previous
import jax
import jax.numpy as jnp
from jax import lax
from jax.experimental import pallas as pl
from jax.experimental.pallas import tpu as pltpu
from jax.experimental.pallas import tpu_sc as plsc

NN = 1000000
EMB = 64
B = 16384
NC, NS, L = 2, 16, 16
NW = NC * NS
SW = 512
NSLAB = 1953
TAIL0 = NSLAB * SW
LIST = 3264
GRP = 512
XCH = 2048
OUTR = 2 * B + 128


def _iota16():
    return lax.iota(jnp.int32, 16)


def _popcnt(m):
    return plsc.all_reduce_population_count(m)[0]


def _sc_body(x1_hbm, x2_hbm, w_hbm, embT_hbm, tail_hbm, out_hbm,
             w_v, xbuf_v, list_v, grpA_v, grpB_v, slabA_v, slabB_v,
             st0, st1, st2, st3, st4, st5, st6, st7,
             semA, semB, ss0, ss1, ss2, ss3, ss4, ss5, ss6, ss7):
    wid = lax.axis_index("s") * NC + lax.axis_index("c")
    it16 = _iota16()
    slab0 = jnp.where(wid == 0, 0, 61 * wid + 1)
    nslab = jnp.where(wid == 0, 62, 61)
    base = slab0 * SW
    limit = nslab * SW + jnp.where(wid == NW - 1, EMB, 0)

    pltpu.sync_copy(w_hbm, w_v)
    z16 = jnp.zeros((L,), jnp.float32)
    stages = (st0, st1, st2, st3, st4, st5, st6, st7)
    ssems = (ss0, ss1, ss2, ss3, ss4, ss5, ss6, ss7)
    junks = tuple(2 * B + 16 * j + it16 for j in range(8))
    for j, st in enumerate(stages):
        for r in range(16):
            for c in range(8):
                st[r, pl.ds(c * 16, L)] = z16
        pltpu.async_copy(st, out_hbm.at[junks[j]], ssems[j])

    def start(s_local, slab_ref, sem):
        src = embT_hbm.at[:, pl.ds((slab0 + s_local) * SW, SW)]
        return pltpu.async_copy(src, slab_ref, sem)

    def wait(s_local, slab_ref, sem):
        src = embT_hbm.at[:, pl.ds((slab0 + s_local) * SW, SW)]
        pltpu.make_async_copy(src, slab_ref, sem).wait()

    start(0, slabA_v, semA)
    start(1, slabB_v, semB)

    def scan_side(x_hbm, side_bit, off0):
        off = off0
        for k in range(B // XCH):
            pltpu.sync_copy(x_hbm.at[pl.ds(k * XCH, XCH)], xbuf_v)

            def sv(v, off, k=k, side_bit=side_bit):
                e = xbuf_v[pl.ds(v * L, L)]
                rel = e - base
                m = (rel >= 0) & (rel < limit)
                pos = k * XCH + v * L + it16
                ent = rel | side_bit | (pos << 16)
                inc = m.astype(jnp.int32)
                c = plsc.cumsum(inc)
                dest = jnp.minimum(off + c - inc, LIST - 1)
                plsc.store_scatter(list_v, [dest], ent, mask=m)
                return off + c[L - 1]

            off = lax.fori_loop(0, XCH // L, sv, off, unroll=4)
        return off

    nlist = scan_side(x2_hbm, 1 << 15, scan_side(x1_hbm, 0, 0))
    sent = jnp.full((L,), 0x7FFF, jnp.int32)

    def pad(v, carry):
        plsc.store_scatter(list_v, [jnp.minimum(nlist + v * L + it16, LIST - 1)],
                           sent, mask=sent > 0)
        return carry

    lax.fori_loop(0, 4, pad, 0)

    def compress_pair(pt):
        def cs(u, offs):
            offA, offB = offs
            for j in range(4):
                v = u * 4 + j
                e = list_v[pl.ds(v * L, L)]
                sid = (e & 0x7FFF) >> 9
                mA = sid == 2 * pt
                mB = sid == 2 * pt + 1
                incA = mA.astype(jnp.int32)
                cA = plsc.cumsum(incA)
                plsc.store_scatter(
                    grpA_v, [jnp.minimum(offA + cA - incA, GRP - 1)], e, mask=mA)
                offA = offA + cA[L - 1]
                incB = mB.astype(jnp.int32)
                cB = plsc.cumsum(incB)
                plsc.store_scatter(
                    grpB_v, [jnp.minimum(offB + cB - incB, GRP - 1)], e, mask=mB)
                offB = offB + cB[L - 1]
            return (offA, offB)

        return lax.fori_loop(0, (nlist + 63) // 64, cs, (0, 0))

    def compress_one(si, grp_ref):
        def cs(u, off):
            for j in range(4):
                v = u * 4 + j
                e = list_v[pl.ds(v * L, L)]
                m = ((e & 0x7FFF) >> 9) == si
                inc = m.astype(jnp.int32)
                c = plsc.cumsum(inc)
                dest = jnp.minimum(off + c - inc, GRP - 1)
                plsc.store_scatter(grp_ref, [dest], e, mask=m)
                off = off + c[L - 1]
            return off

        return lax.fori_loop(0, (nlist + 63) // 64, cs, 0)

    def do_builds(cnt, grp_ref, slab_ref, col_off, bufset):
        ngrp = (cnt + L - 1) // L

        def build(g, stage_ref, ssem, jrow):
            eg = grp_ref[pl.ds(g * L, L)]
            valid = (g * L + it16) < cnt
            col = jnp.where(valid, (eg & 511) + col_off, 0)
            side = (eg >> 15) & 1
            pos = jnp.where(valid, eg >> 16, 0)
            wv = plsc.load_gather(w_v, [pos])
            wsel = jnp.where((side == 0) & valid, wv, jnp.ones((L,), jnp.float32))
            pltpu.make_async_copy(stage_ref, out_hbm.at[jrow], ssem).wait()

            def dblk(i, carry):
                for j in range(8):
                    dvec = jnp.full((L,), 0, jnp.int32) + (i * 8 + j)
                    fv = plsc.load_gather(slab_ref, [dvec, col])
                    plsc.store_scatter(stage_ref, [it16, dvec], fv * wsel)
                return carry

            lax.fori_loop(0, EMB // 8, dblk, 0)
            drow = jnp.where(valid, pos + side * B, jrow)
            pltpu.async_copy(stage_ref, out_hbm.at[drow], ssem)

        def gquad(t, carry):
            build(4 * t, stages[bufset], ssems[bufset], junks[bufset])
            for j in range(1, 4):
                @pl.when(4 * t + j < ngrp)
                def _(j=j):
                    build(4 * t + j, stages[bufset + j],
                          ssems[bufset + j], junks[bufset + j])
            return carry

        lax.fori_loop(0, (ngrp + 3) // 4, gquad, 0)

    def compute_slab(si, slab_ref, col_off, bufset):
        do_builds(compress_one(si, grpA_v), grpA_v, slab_ref, col_off, bufset)

    def pair(t, carry):
        s0 = 2 * t
        cntA, cntB = compress_pair(t)
        wait(s0, slabA_v, semA)
        do_builds(cntA, grpA_v, slabA_v, 0, 0)

        @pl.when(s0 + 2 < nslab)
        def _():
            start(s0 + 2, slabA_v, semA)

        @pl.when(s0 + 1 < nslab)
        def _():
            wait(s0 + 1, slabB_v, semB)
            do_builds(cntB, grpB_v, slabB_v, 0, 4)

            @pl.when(s0 + 3 < nslab)
            def _():
                start(s0 + 3, slabB_v, semB)

        return carry

    lax.fori_loop(0, 31, pair, 0)

    @pl.when(wid == NW - 1)
    def _():
        pltpu.sync_copy(tail_hbm, slabA_v.at[:, pl.ds(0, 2 * EMB)])
        compute_slab(61, slabA_v, EMB, 0)

    for j in range(8):
        pltpu.make_async_copy(stages[j], out_hbm.at[junks[j]], ssems[j]).wait()


def _gather_rows(x1, x2, w, emb):
    embT = emb.T
    tail = lax.slice(embT, (0, NN - 2 * EMB), (EMB, NN))
    mesh = plsc.VectorSubcoreMesh(core_axis_name="c", subcore_axis_name="s")
    return pl.kernel(
        _sc_body,
        out_type=jax.ShapeDtypeStruct((OUTR, 2 * EMB), jnp.float32),
        mesh=mesh,
        compiler_params=pltpu.CompilerParams(needs_layout_passes=False),
        scratch_types=[
            pltpu.VMEM((B,), jnp.float32),
            pltpu.VMEM((XCH,), jnp.int32),
            pltpu.VMEM((LIST,), jnp.int32),
            pltpu.VMEM((GRP,), jnp.int32),
            pltpu.VMEM((GRP,), jnp.int32),
            pltpu.VMEM((EMB, SW), jnp.float32),
            pltpu.VMEM((EMB, SW), jnp.float32),
        ] + [pltpu.VMEM((L, 2 * EMB), jnp.float32)] * 8
          + [pltpu.SemaphoreType.DMA] * 10,
    )(x1, x2, w, embT, tail)


def _tc_body(p_ref, o_ref):
    e1 = p_ref[pl.ds(0, B), :]
    e2 = p_ref[pl.ds(B, B), :]
    x = jnp.sum(e1 * e2, axis=1, keepdims=True)
    ls = jnp.minimum(x, 0.0) - jnp.log1p(jnp.exp(-jnp.abs(x)))
    o_ref[0, 0] = -jnp.sum(ls) / B


def _loss(rows):
    return pl.pallas_call(
        _tc_body,
        out_shape=jax.ShapeDtypeStruct((1, 1), jnp.float32),
        in_specs=[pl.BlockSpec(memory_space=pltpu.VMEM)],
        out_specs=pl.BlockSpec(memory_space=pltpu.SMEM),
    )(rows)


@jax.jit
def kernel(x1, x2, w, emb):
    return _loss(_gather_rows(x1, x2, w, emb))[0, 0]


# speedup vs baseline: 1.2184x; 1.0007x over previous
import jax
import jax.numpy as jnp
from jax import lax
from jax.experimental import pallas as pl
from jax.experimental.pallas import tpu as pltpu
from jax.experimental.pallas import tpu_sc as plsc

NN = 1000000
EMB = 64
B = 16384
NC, NS, L = 2, 16, 16
NW = NC * NS
SW = 512
NSLAB = 1953
TAIL0 = NSLAB * SW
LIST = 3264
GRP = 512
XCH = 2048
OUTR = 2 * B + 128


def _iota16():
    return lax.iota(jnp.int32, 16)


def _popcnt(m):
    return plsc.all_reduce_population_count(m)[0]


def _sc_body(x1_hbm, x2_hbm, w_hbm, embT_hbm, tail_hbm, out_hbm,
             w_v, xbuf_v, list_v, grpA_v, grpB_v, slabA_v, slabB_v,
             st0, st1, st2, st3, st4, st5, st6, st7,
             semA, semB, ss0, ss1, ss2, ss3, ss4, ss5, ss6, ss7):
    wid = lax.axis_index("s") * NC + lax.axis_index("c")
    it16 = _iota16()
    slab0 = jnp.where(wid == 0, 0, 61 * wid + 1)
    nslab = jnp.where(wid == 0, 62, 61)
    base = slab0 * SW
    limit = nslab * SW + jnp.where(wid == NW - 1, EMB, 0)

    pltpu.sync_copy(w_hbm, w_v)
    z16 = jnp.zeros((L,), jnp.float32)
    stages = (st0, st1, st2, st3, st4, st5, st6, st7)
    ssems = (ss0, ss1, ss2, ss3, ss4, ss5, ss6, ss7)
    junks = tuple(2 * B + 16 * j + it16 for j in range(8))
    for j, st in enumerate(stages):
        for r in range(16):
            for c in range(8):
                st[r, pl.ds(c * 16, L)] = z16
        pltpu.async_copy(st, out_hbm.at[junks[j]], ssems[j])

    def start(s_local, slab_ref, sem):
        src = embT_hbm.at[:, pl.ds((slab0 + s_local) * SW, SW)]
        return pltpu.async_copy(src, slab_ref, sem)

    def wait(s_local, slab_ref, sem):
        src = embT_hbm.at[:, pl.ds((slab0 + s_local) * SW, SW)]
        pltpu.make_async_copy(src, slab_ref, sem).wait()

    start(0, slabA_v, semA)
    start(1, slabB_v, semB)

    def scan_side(x_hbm, side_bit, off0):
        off = off0
        for k in range(B // XCH):
            pltpu.sync_copy(x_hbm.at[pl.ds(k * XCH, XCH)], xbuf_v)

            def sv(v, off, k=k, side_bit=side_bit):
                e = xbuf_v[pl.ds(v * L, L)]
                rel = e - base
                m = (rel >= 0) & (rel < limit)
                pos = k * XCH + v * L + it16
                ent = rel | side_bit | (pos << 16)
                inc = m.astype(jnp.int32)
                c = plsc.cumsum(inc)
                dest = jnp.minimum(off + c - inc, LIST - 1)
                plsc.store_scatter(list_v, [dest], ent, mask=m)
                return off + c[L - 1]

            off = lax.fori_loop(0, XCH // L, sv, off, unroll=4)
        return off

    nlist = scan_side(x2_hbm, 1 << 15, scan_side(x1_hbm, 0, 0))
    sent = jnp.full((L,), 0x7FFF, jnp.int32)

    def pad(v, carry):
        plsc.store_scatter(list_v, [jnp.minimum(nlist + v * L + it16, LIST - 1)],
                           sent, mask=sent > 0)
        return carry

    lax.fori_loop(0, 4, pad, 0)

    def compress_pair(pt):
        def cs(u, offs):
            offA, offB = offs
            for j in range(4):
                v = u * 4 + j
                e = list_v[pl.ds(v * L, L)]
                sid = (e & 0x7FFF) >> 9
                mA = sid == 2 * pt
                mB = sid == 2 * pt + 1
                incA = mA.astype(jnp.int32)
                cA = plsc.cumsum(incA)
                plsc.store_scatter(
                    grpA_v, [jnp.minimum(offA + cA - incA, GRP - 1)], e, mask=mA)
                offA = offA + cA[L - 1]
                incB = mB.astype(jnp.int32)
                cB = plsc.cumsum(incB)
                plsc.store_scatter(
                    grpB_v, [jnp.minimum(offB + cB - incB, GRP - 1)], e, mask=mB)
                offB = offB + cB[L - 1]
            return (offA, offB)

        return lax.fori_loop(0, (nlist + 63) // 64, cs, (0, 0))

    def compress_one(si, grp_ref):
        def cs(u, off):
            for j in range(4):
                v = u * 4 + j
                e = list_v[pl.ds(v * L, L)]
                m = ((e & 0x7FFF) >> 9) == si
                inc = m.astype(jnp.int32)
                c = plsc.cumsum(inc)
                dest = jnp.minimum(off + c - inc, GRP - 1)
                plsc.store_scatter(grp_ref, [dest], e, mask=m)
                off = off + c[L - 1]
            return off

        return lax.fori_loop(0, (nlist + 63) // 64, cs, 0)

    def do_builds(cnt, grp_ref, slab_ref, col_off, bufset):
        ngrp = (cnt + L - 1) // L

        def build(g, stage_ref, ssem, jrow):
            eg = grp_ref[pl.ds(g * L, L)]
            valid = (g * L + it16) < cnt
            col = jnp.where(valid, (eg & 511) + col_off, 0)
            side = (eg >> 15) & 1
            pos = jnp.where(valid, eg >> 16, 0)
            wv = plsc.load_gather(w_v, [pos])
            wsel = jnp.where((side == 0) & valid, wv, jnp.ones((L,), jnp.float32))
            pltpu.make_async_copy(stage_ref, out_hbm.at[jrow], ssem).wait()

            def dblk(i, carry):
                for j in range(8):
                    dvec = jnp.full((L,), 0, jnp.int32) + (i * 8 + j)
                    fv = plsc.load_gather(slab_ref, [dvec, col])
                    plsc.store_scatter(stage_ref, [it16, dvec], fv * wsel)
                return carry

            lax.fori_loop(0, EMB // 8, dblk, 0)
            drow = jnp.where(valid, pos + side * B, jrow)
            pltpu.async_copy(stage_ref, out_hbm.at[drow], ssem)

        def gquad(t, carry):
            build(4 * t, stages[bufset], ssems[bufset], junks[bufset])
            for j in range(1, 4):
                @pl.when(4 * t + j < ngrp)
                def _(j=j):
                    build(4 * t + j, stages[bufset + j],
                          ssems[bufset + j], junks[bufset + j])
            return carry

        lax.fori_loop(0, (ngrp + 3) // 4, gquad, 0)

    def compute_slab(si, slab_ref, col_off, bufset):
        do_builds(compress_one(si, grpA_v), grpA_v, slab_ref, col_off, bufset)

    def pair(t, carry):
        s0 = 2 * t
        cntA, cntB = compress_pair(t)
        wait(s0, slabA_v, semA)
        do_builds(cntA, grpA_v, slabA_v, 0, 0)

        @pl.when(s0 + 2 < nslab)
        def _():
            start(s0 + 2, slabA_v, semA)

        @pl.when(s0 + 1 < nslab)
        def _():
            wait(s0 + 1, slabB_v, semB)
            do_builds(cntB, grpB_v, slabB_v, 0, 4)

            @pl.when(s0 + 3 < nslab)
            def _():
                start(s0 + 3, slabB_v, semB)

        return carry

    lax.fori_loop(0, 31, pair, 0)

    @pl.when(wid == NW - 1)
    def _():
        pltpu.sync_copy(tail_hbm, slabA_v.at[:, pl.ds(0, 2 * EMB)])
        compute_slab(61, slabA_v, EMB, 0)

    for j in range(8):
        pltpu.make_async_copy(stages[j], out_hbm.at[junks[j]], ssems[j]).wait()


def _gather_rows(x1, x2, w, emb):
    embT = emb.T
    tail = lax.slice(embT, (0, NN - 2 * EMB), (EMB, NN))
    mesh = plsc.VectorSubcoreMesh(core_axis_name="c", subcore_axis_name="s")
    return pl.kernel(
        _sc_body,
        out_type=jax.ShapeDtypeStruct((OUTR, 2 * EMB), jnp.float32),
        mesh=mesh,
        compiler_params=pltpu.CompilerParams(needs_layout_passes=False),
        scratch_types=[
            pltpu.VMEM((B,), jnp.float32),
            pltpu.VMEM((XCH,), jnp.int32),
            pltpu.VMEM((LIST,), jnp.int32),
            pltpu.VMEM((GRP,), jnp.int32),
            pltpu.VMEM((GRP,), jnp.int32),
            pltpu.VMEM((EMB, SW), jnp.float32),
            pltpu.VMEM((EMB, SW), jnp.float32),
        ] + [pltpu.VMEM((L, 2 * EMB), jnp.float32)] * 8
          + [pltpu.SemaphoreType.DMA] * 10,
    )(x1, x2, w, embT, tail)


def _tc_body(p_ref, o_ref):
    e1 = p_ref[pl.ds(0, B), :]
    e2 = p_ref[pl.ds(B, B), :]
    x = jnp.sum(e1 * e2, axis=1)
    xr = x.reshape(2 * EMB, 2 * EMB)
    ls = jnp.minimum(xr, 0.0) - jnp.log1p(jnp.exp(-jnp.abs(xr)))
    o_ref[0, 0] = -jnp.sum(ls) / B


def _loss(rows):
    return pl.pallas_call(
        _tc_body,
        out_shape=jax.ShapeDtypeStruct((1, 1), jnp.float32),
        in_specs=[pl.BlockSpec(memory_space=pltpu.VMEM)],
        out_specs=pl.BlockSpec(memory_space=pltpu.SMEM),
    )(rows)


@jax.jit
def kernel(x1, x2, w, emb):
    return _loss(_gather_rows(x1, x2, w, emb))[0, 0]
